# baseline (speedup 1.0000x reference)
import jax
import jax.numpy as jnp
from jax import lax
from jax.experimental import pallas as pl
from jax.experimental.pallas import tpu as pltpu
from jax.experimental.pallas import tpu_sc as plsc

NC = 2
NS = 16
L = 16
NW = NC * NS

B = 32
EC = 2000


def _matmul_kernel(x_ref, w_ref, o_ref):
    o_ref[...] = jnp.dot(x_ref[...], w_ref[...],
                         preferred_element_type=jnp.float32)


def _matmul(x, W):
    n, din = x.shape
    dout = W.shape[1]
    bm = 1000
    return pl.pallas_call(
        _matmul_kernel,
        grid=(n // bm,),
        in_specs=[
            pl.BlockSpec((bm, din), lambda i: (i, 0)),
            pl.BlockSpec((din, dout), lambda i: (0, 0)),
        ],
        out_specs=pl.BlockSpec((bm, dout), lambda i: (i, 0)),
        out_shape=jax.ShapeDtypeStruct((n, dout), jnp.float32),
    )(x, W)


def _make_sc_scatter(n, e, d):
    nper = (n + NW - 1) // NW
    npad = ((nper + 63) // 64) * 64
    cap = EC + B
    nq = d // L
    nchunk = e // EC

    mesh = plsc.VectorSubcoreMesh(core_axis_name="c", subcore_axis_name="s")

    def body(presup, srch, dsth, ewh, out,
             src_a, dst_a, w_a, src_b, dst_b, w_b,
             csrc, cw, cdst, rows2, acc, sem, sem2):
        c = lax.axis_index("c")
        s = lax.axis_index("s")
        w = c * NS + s
        lo = w * nper
        zf = jnp.zeros((L,), jnp.float32)
        zi = jnp.zeros((L,), jnp.int32)
        onei = jnp.ones((L,), jnp.int32)

        def zrow(i, _):
            for q in range(nq):
                acc[i, pl.ds(q * L, L)] = zf
            return 0
        lax.fori_loop(0, npad, zrow, 0)

        def mk_comp(sref, dref, wref):
            def comp_(i, cnt):
                lo_v = jnp.full((L,), lo, jnp.int32)
                off = i * L
                dv = dref[pl.ds(off, L)]
                sv = sref[pl.ds(off, L)]
                wv = wref[pl.ds(off, L)]
                m = (dv >= lo_v) & (dv < lo_v + nper)
                mi = jnp.where(m, onei, zi)
                pos = jnp.full((L,), cnt, jnp.int32) + plsc.cumsum(mi) - 1
                plsc.store_scatter(csrc, [pos], sv, mask=m)
                plsc.store_scatter(cw, [pos], wv, mask=m)
                plsc.store_scatter(cdst, [pos], dv - lo_v, mask=m)
                return pos[L - 1] + 1
            return comp_

        def stage(t, sref, dref, wref):
            pltpu.async_copy(srch.at[pl.ds(t * EC, EC)], sref, sem2)
            pltpu.async_copy(dsth.at[pl.ds(t * EC, EC)], dref, sem2)
            pltpu.async_copy(ewh.at[pl.ds(t * EC, EC)], wref, sem2)

        def stage_wait(t, sref, dref, wref):
            pltpu.make_async_copy(srch.at[pl.ds(t * EC, EC)], sref,
                                  sem2).wait()
            pltpu.make_async_copy(dsth.at[pl.ds(t * EC, EC)], dref,
                                  sem2).wait()
            pltpu.make_async_copy(ewh.at[pl.ds(t * EC, EC)], wref,
                                  sem2).wait()

        def process(sref, dref, wref):
            cnt = lax.fori_loop(0, EC // L, mk_comp(sref, dref, wref),
                                jnp.int32(0))
            cnt_v = jnp.full((L,), cnt, jnp.int32)
            nb = (cnt + (B - 1)) // B
            lanes = lax.iota(jnp.int32, L)
            lim = jnp.full((L,), nb * B, jnp.int32)
            for j in range(B // L):
                pos = cnt_v + lanes + (j * L)
                mp = pos < lim
                plsc.store_scatter(csrc, [pos], zi, mask=mp)
                plsc.store_scatter(cw, [pos], zf, mask=mp)
                plsc.store_scatter(cdst, [pos], zi, mask=mp)

            @pl.when(nb > 0)
            def _():
                pltpu.async_copy(presup.at[csrc.at[pl.ds(0, B)]],
                                 rows2.at[0], sem)

            def run(b, _):
                par = jnp.bitwise_and(b, 1)

                @pl.when(b + 1 < nb)
                def _():
                    pltpu.async_copy(
                        presup.at[csrc.at[pl.ds((b + 1) * B, B)]],
                        rows2.at[jnp.bitwise_and(b + 1, 1)], sem)

                pltpu.make_async_copy(presup.at[csrc.at[pl.ds(b * B, B)]],
                                      rows2.at[par], sem).wait()

                def fma(i16, _):
                    base = b * B + i16 * L
                    wv = cw[pl.ds(base, L)]
                    dlv = cdst[pl.ds(base, L)]
                    for r in range(L):
                        i = i16 * L + r
                        wsc = wv[r]
                        dl = dlv[r]
                        for q in range(nq):
                            sl = pl.ds(q * L, L)
                            plsc.addupdate(acc.at[dl, sl],
                                           rows2[par, i, sl] * wsc)
                    return 0
                lax.fori_loop(0, B // L, fma, 0)
                return 0
            lax.fori_loop(0, nb, run, 0)

        stage(0, src_a, dst_a, w_a)

        def pair(t2, _):
            t = t2 * 2
            stage(t + 1, src_b, dst_b, w_b)
            stage_wait(t, src_a, dst_a, w_a)
            process(src_a, dst_a, w_a)

            @pl.when(t + 2 < nchunk)
            def _():
                stage(t + 2, src_a, dst_a, w_a)

            stage_wait(t + 1, src_b, dst_b, w_b)
            process(src_b, dst_b, w_b)
            return 0
        lax.fori_loop(0, nchunk // 2, pair, 0)

        def relu_row(i, _):
            for q in range(nq):
                sl = pl.ds(q * L, L)
                acc[i, sl] = jnp.maximum(acc[i, sl], zf)
            return 0
        lax.fori_loop(0, npad, relu_row, 0)
        for k in range(npad // 64):
            pltpu.sync_copy(acc.at[pl.ds(k * 64, 64)],
                            out.at[pl.ds(w * npad + k * 64, 64)])

    return pl.kernel(
        body,
        out_type=jax.ShapeDtypeStruct((NW * npad, d), jnp.float32),
        mesh=mesh,
        compiler_params=pltpu.CompilerParams(needs_layout_passes=False),
        scratch_types=[
            pltpu.VMEM((EC,), jnp.int32),
            pltpu.VMEM((EC,), jnp.int32),
            pltpu.VMEM((EC,), jnp.float32),
            pltpu.VMEM((EC,), jnp.int32),
            pltpu.VMEM((EC,), jnp.int32),
            pltpu.VMEM((EC,), jnp.float32),
            pltpu.VMEM((cap,), jnp.int32),
            pltpu.VMEM((cap,), jnp.float32),
            pltpu.VMEM((cap,), jnp.int32),
            pltpu.VMEM((2, B, d), jnp.float32),
            pltpu.VMEM((npad, d), jnp.float32),
            pltpu.SemaphoreType.DMA,
            pltpu.SemaphoreType.DMA,
        ],
    ), npad, nper


def kernel(x, edge_index, edge_weight, W):
    n, din = x.shape
    e = edge_index.shape[1]
    dout = W.shape[1]

    pre_sup = _matmul(x, W)
    src = edge_index[0]
    dst = edge_index[1]

    sc, npad, nper = _make_sc_scatter(n, e, dout)
    out_pad = sc(pre_sup, src, dst, edge_weight)
    out = out_pad.reshape(NW, npad, dout)[:, :nper, :].reshape(NW * nper,
                                                               dout)[:n]
    return out


# speedup vs baseline: 1.8392x; 1.8392x over previous
import jax
import jax.numpy as jnp
from jax import lax
from jax.experimental import pallas as pl
from jax.experimental.pallas import tpu as pltpu
from jax.experimental.pallas import tpu_sc as plsc

NC = 2
NS = 16
L = 16
NW = NC * NS

B = 16
NBUF = 4
EC = 3200


def _matmul_kernel(x_ref, w_ref, o_ref):
    o_ref[...] = jnp.dot(x_ref[...], w_ref[...],
                         preferred_element_type=jnp.float32)


def _matmul(x, W):
    n, din = x.shape
    dout = W.shape[1]
    bm = 1000
    return pl.pallas_call(
        _matmul_kernel,
        grid=(n // bm,),
        in_specs=[
            pl.BlockSpec((bm, din), lambda i: (i, 0)),
            pl.BlockSpec((din, dout), lambda i: (0, 0)),
        ],
        out_specs=pl.BlockSpec((bm, dout), lambda i: (i, 0)),
        out_shape=jax.ShapeDtypeStruct((n, dout), jnp.float32),
    )(x, W)


def _make_sc_scatter(n, e, d):
    nper = (n + NW - 1) // NW
    npad = ((nper + 63) // 64) * 64
    cap = EC + B
    nq = d // L
    nchunk = e // EC

    mesh = plsc.VectorSubcoreMesh(core_axis_name="c", subcore_axis_name="s")

    def body(presup, srch, dsth, ewh, out,
             src_v, dst_v, w_v, csrc, cw, cdst, rows2, acc, sem):
        c = lax.axis_index("c")
        s = lax.axis_index("s")
        w = c * NS + s
        lo = w * nper
        zf = jnp.zeros((L,), jnp.float32)
        zi = jnp.zeros((L,), jnp.int32)
        onei = jnp.ones((L,), jnp.int32)

        def zrow(i, _):
            for q in range(nq):
                acc[i, pl.ds(q * L, L)] = zf
            return 0
        lax.fori_loop(0, npad, zrow, 0)

        def comp(i, cnt):
            lo_v = jnp.full((L,), lo, jnp.int32)
            off = i * L
            dv = dst_v[pl.ds(off, L)]
            sv = src_v[pl.ds(off, L)]
            wv = w_v[pl.ds(off, L)]
            m = (dv >= lo_v) & (dv < lo_v + nper)
            mi = jnp.where(m, onei, zi)
            pos = jnp.full((L,), cnt, jnp.int32) + plsc.cumsum(mi) - 1
            plsc.store_scatter(csrc, [pos], sv, mask=m)
            plsc.store_scatter(cw, [pos], wv, mask=m)
            plsc.store_scatter(cdst, [pos], dv - lo_v, mask=m)
            return pos[L - 1] + 1

        def chunk(t, _):
            pltpu.sync_copy(srch.at[pl.ds(t * EC, EC)], src_v)
            pltpu.sync_copy(dsth.at[pl.ds(t * EC, EC)], dst_v)
            pltpu.sync_copy(ewh.at[pl.ds(t * EC, EC)], w_v)
            cnt = lax.fori_loop(0, EC // L, comp, jnp.int32(0))
            cnt_v = jnp.full((L,), cnt, jnp.int32)
            nb = (cnt + (B - 1)) // B
            lanes = lax.iota(jnp.int32, L)
            lim = jnp.full((L,), nb * B, jnp.int32)
            for j in range(B // L):
                pos = cnt_v + lanes + (j * L)
                mp = pos < lim
                plsc.store_scatter(csrc, [pos], zi, mask=mp)
                plsc.store_scatter(cw, [pos], zf, mask=mp)
                plsc.store_scatter(cdst, [pos], zi, mask=mp)

            for k in range(NBUF):
                @pl.when(k < nb)
                def _():
                    pltpu.async_copy(presup.at[csrc.at[pl.ds(k * B, B)]],
                                     rows2.at[k], sem)

            def run(b, _):
                par = jnp.bitwise_and(b, NBUF - 1)

                pltpu.make_async_copy(presup.at[csrc.at[pl.ds(b * B, B)]],
                                      rows2.at[par], sem).wait()

                def fma(i16, _):
                    base = b * B + i16 * L
                    wv = cw[pl.ds(base, L)]
                    dlv = cdst[pl.ds(base, L)]
                    for r in range(L):
                        i = i16 * L + r
                        wsc = wv[r]
                        dl = dlv[r]
                        for q in range(nq):
                            sl = pl.ds(q * L, L)
                            plsc.addupdate(acc.at[dl, sl],
                                           rows2[par, i, sl] * wsc)
                    return 0
                lax.fori_loop(0, B // L, fma, 0)

                @pl.when(b + NBUF < nb)
                def _():
                    pltpu.async_copy(
                        presup.at[csrc.at[pl.ds((b + NBUF) * B, B)]],
                        rows2.at[par], sem)
                return 0
            lax.fori_loop(0, nb, run, 0)
            return 0
        lax.fori_loop(0, nchunk, chunk, 0)

        def relu_row(i, _):
            for q in range(nq):
                sl = pl.ds(q * L, L)
                acc[i, sl] = jnp.maximum(acc[i, sl], zf)
            return 0
        lax.fori_loop(0, npad, relu_row, 0)
        for k in range(npad // 64):
            pltpu.sync_copy(acc.at[pl.ds(k * 64, 64)],
                            out.at[pl.ds(w * npad + k * 64, 64)])

    return pl.kernel(
        body,
        out_type=jax.ShapeDtypeStruct((NW * npad, d), jnp.float32),
        mesh=mesh,
        compiler_params=pltpu.CompilerParams(needs_layout_passes=False),
        scratch_types=[
            pltpu.VMEM((EC,), jnp.int32),
            pltpu.VMEM((EC,), jnp.int32),
            pltpu.VMEM((EC,), jnp.float32),
            pltpu.VMEM((cap,), jnp.int32),
            pltpu.VMEM((cap,), jnp.float32),
            pltpu.VMEM((cap,), jnp.int32),
            pltpu.VMEM((NBUF, B, d), jnp.float32),
            pltpu.VMEM((npad, d), jnp.float32),
            pltpu.SemaphoreType.DMA,
        ],
    ), npad, nper


def kernel(x, edge_index, edge_weight, W):
    n, din = x.shape
    e = edge_index.shape[1]
    dout = W.shape[1]

    pre_sup = _matmul(x, W)
    src = edge_index[0]
    dst = edge_index[1]

    sc, npad, nper = _make_sc_scatter(n, e, dout)
    out_pad = sc(pre_sup, src, dst, edge_weight)
    out = out_pad.reshape(NW, npad, dout)[:, :nper, :].reshape(NW * nper,
                                                               dout)[:n]
    return out
